# baseline (speedup 1.0000x reference)
import functools

import jax
import jax.numpy as jnp
from jax import lax
from jax.experimental import pallas as pl
from jax.experimental.pallas import tpu as pltpu
from jax.experimental.pallas import tpu_sc as plsc

NC = 2
NS = 16
NW = NC * NS
CHUNK = 80
NPAD = 10240
D = 128


def _sc_mesh():
    return plsc.VectorSubcoreMesh(
        core_axis_name="c", subcore_axis_name="s", num_cores=NC, num_subcores=NS
    )


DDEG = 128


def _make_deg(e):
    ept = e // NW
    nchunk = ept // CHUNK
    rpt = NPAD // NS

    @functools.partial(
        pl.kernel,
        out_type=jax.ShapeDtypeStruct((NC, NPAD, DDEG), jnp.float32),
        mesh=_sc_mesh(),
        scratch_types=[
            pltpu.VMEM((nchunk, CHUNK), jnp.int32),
            pltpu.VMEM((CHUNK, DDEG), jnp.float32),
            pltpu.VMEM_SHARED((NPAD, DDEG), jnp.float32),
        ],
    )
    def deg_kernel(dst_hbm, ones_hbm, zeros_hbm, out_hbm, didx_v, ones_v, acc_sh):
        c = lax.axis_index("c")
        s = lax.axis_index("s")
        wid = c * NS + s
        base_r = s * rpt
        pltpu.sync_copy(zeros_hbm, acc_sh.at[pl.ds(base_r, rpt)])
        pltpu.sync_copy(ones_hbm, ones_v)
        pltpu.sync_copy(dst_hbm.at[wid], didx_v)
        plsc.subcore_barrier()

        def body(i, carry):
            pltpu.sync_copy(ones_v, acc_sh.at[didx_v.at[i]], add=True)
            return carry

        lax.fori_loop(0, nchunk, body, 0)
        plsc.subcore_barrier()
        pltpu.sync_copy(acc_sh.at[pl.ds(base_r, rpt)], out_hbm.at[c, pl.ds(base_r, rpt)])

    return deg_kernel


def _make_agg(e, d):
    ept = e // NW
    nchunk = ept // CHUNK
    rpt = NPAD // NS

    @functools.partial(
        pl.kernel,
        out_type=jax.ShapeDtypeStruct((NC, NPAD, d), jnp.float32),
        mesh=_sc_mesh(),
        scratch_types=[
            pltpu.VMEM((2, CHUNK), jnp.int32),
            pltpu.VMEM((nchunk, CHUNK), jnp.int32),
            pltpu.VMEM((2, CHUNK, d), jnp.float32),
            pltpu.VMEM_SHARED((NPAD, d), jnp.float32),
            pltpu.SemaphoreType.DMA,
            pltpu.SemaphoreType.DMA,
        ],
    )
    def agg_kernel(h_hbm, src_hbm, dst_hbm, zeros_hbm, out_hbm,
                   sidx_v, didx_v, rows_v, acc_sh, sem0, sem1):
        c = lax.axis_index("c")
        s = lax.axis_index("s")
        wid = c * NS + s
        base_r = s * rpt
        pltpu.sync_copy(zeros_hbm, acc_sh.at[pl.ds(base_r, rpt)])
        pltpu.sync_copy(dst_hbm.at[wid], didx_v)
        plsc.subcore_barrier()

        pltpu.sync_copy(src_hbm.at[wid, 0], sidx_v.at[0])
        pltpu.async_copy(h_hbm.at[sidx_v.at[0]], rows_v.at[0], sem0)

        def body(p, carry):
            i0 = 2 * p
            pltpu.sync_copy(src_hbm.at[wid, i0 + 1], sidx_v.at[1])
            pltpu.async_copy(h_hbm.at[sidx_v.at[1]], rows_v.at[1], sem1)
            pltpu.make_async_copy(h_hbm.at[sidx_v.at[0]], rows_v.at[0], sem0).wait()
            pltpu.sync_copy(rows_v.at[0], acc_sh.at[didx_v.at[i0]], add=True)

            @pl.when(i0 + 2 < nchunk)
            def _():
                pltpu.sync_copy(src_hbm.at[wid, i0 + 2], sidx_v.at[0])
                pltpu.async_copy(h_hbm.at[sidx_v.at[0]], rows_v.at[0], sem0)

            pltpu.make_async_copy(h_hbm.at[sidx_v.at[1]], rows_v.at[1], sem1).wait()
            pltpu.sync_copy(rows_v.at[1], acc_sh.at[didx_v.at[i0 + 1]], add=True)
            return carry

        lax.fori_loop(0, nchunk // 2, body, 0)
        if nchunk % 2 == 1:
            last = nchunk - 1
            pltpu.make_async_copy(h_hbm.at[sidx_v.at[0]], rows_v.at[0], sem0).wait()
            pltpu.sync_copy(rows_v.at[0], acc_sh.at[didx_v.at[last]], add=True)
        plsc.subcore_barrier()
        pltpu.sync_copy(acc_sh.at[pl.ds(base_r, rpt)], out_hbm.at[c, pl.ds(base_r, rpt)])

    return agg_kernel


def _tc_mm(x, W1):
    n = x.shape[0]

    def body(x_ref, w_ref, m_ref):
        m_ref[...] = jnp.dot(x_ref[...], w_ref[...], preferred_element_type=jnp.float32)

    return pl.pallas_call(
        body, out_shape=jax.ShapeDtypeStruct((n, W1.shape[1]), jnp.float32)
    )(x, W1)


def _tc_scale(cnt, m):
    n = m.shape[0]

    def body(cnt_ref, m_ref, h_ref, dis_ref):
        deg = 1.0 + cnt_ref[0, :n, 0:1] + cnt_ref[1, :n, 0:1]
        dis = lax.rsqrt(deg)
        dis_ref[...] = dis
        h_ref[...] = m_ref[...] * dis

    return pl.pallas_call(
        body,
        out_shape=(
            jax.ShapeDtypeStruct((n, m.shape[1]), jnp.float32),
            jax.ShapeDtypeStruct((n, 1), jnp.float32),
        ),
    )(cnt, m)


def _tc_layer(p, hp, dis, b, g, be, Wn):
    n, _ = hp.shape
    dn = Wn.shape[1]

    def body(p_ref, hp_ref, dis_ref, b_ref, g_ref, be_ref, w_ref, o_ref):
        t = (p_ref[0, :n] + p_ref[1, :n] + hp_ref[...]) * dis_ref[...] + b_ref[...]
        mu = jnp.mean(t, axis=0, keepdims=True)
        dt = t - mu
        var = jnp.mean(dt * dt, axis=0, keepdims=True)
        z = g_ref[...] * dt * lax.rsqrt(var + 1e-5) + be_ref[...]
        z = jnp.maximum(z, 0.0)
        o_ref[...] = jnp.dot(z, w_ref[...], preferred_element_type=jnp.float32) * dis_ref[...]

    return pl.pallas_call(
        body, out_shape=jax.ShapeDtypeStruct((n, dn), jnp.float32)
    )(p, hp, dis, b, g, be, Wn)


def _tc_final(p, hp, dis, b, dout):
    n, _ = hp.shape

    def body(p_ref, hp_ref, dis_ref, b_ref, o_ref):
        t = (p_ref[0, :n] + p_ref[1, :n] + hp_ref[...]) * dis_ref[...] + b_ref[...]
        u = t[:, :dout]
        m = jnp.max(u, axis=1, keepdims=True)
        ex = jnp.exp(u - m)
        lse = jnp.log(jnp.sum(ex, axis=1, keepdims=True))
        o_ref[...] = u - m - lse

    return pl.pallas_call(
        body, out_shape=jax.ShapeDtypeStruct((n, dout), jnp.float32)
    )(p, hp, dis, b)


def kernel(x, edge_index, W1, b1, g1, be1, W2, b2, g2, be2, W3, b3):
    n = x.shape[0]
    e = edge_index.shape[1]
    dout = W3.shape[1]
    ept = e // NW
    nchunk = ept // CHUNK

    d3 = D
    ei = edge_index.astype(jnp.int32)
    src3 = ei[0].reshape(NW, nchunk, CHUNK)
    dst3 = ei[1].reshape(NW, nchunk, CHUNK)
    W3p = jnp.pad(W3, ((0, 0), (0, d3 - dout)))
    b3p = jnp.pad(b3, (0, d3 - dout))
    zrows = jnp.zeros((NPAD // NS, D), jnp.float32)
    zrows16 = jnp.zeros((NPAD // NS, DDEG), jnp.float32)
    ones16 = jnp.ones((CHUNK, DDEG), jnp.float32)

    cnt = _make_deg(e)(dst3, ones16, zrows16)
    m1 = _tc_mm(x, W1)
    h1p, dis = _tc_scale(cnt, m1)

    agg = _make_agg(e, D)
    p1 = agg(h1p, src3, dst3, zrows)
    h2p = _tc_layer(p1, h1p, dis, b1[None], g1[None], be1[None], W2)
    p2 = agg(h2p, src3, dst3, zrows)
    h3p = _tc_layer(p2, h2p, dis, b2[None], g2[None], be2[None], W3p)
    p3 = agg(h3p, src3, dst3, zrows)
    return _tc_final(p3, h3p, dis, b3p[None], dout)


# speedup vs baseline: 1.1087x; 1.1087x over previous
import functools

import jax
import jax.numpy as jnp
from jax import lax
from jax.experimental import pallas as pl
from jax.experimental.pallas import tpu as pltpu
from jax.experimental.pallas import tpu_sc as plsc

NC = 2
NS = 16
NW = NC * NS
CHUNK = 80
NPAD = 10240
D = 128


def _sc_mesh():
    return plsc.VectorSubcoreMesh(
        core_axis_name="c", subcore_axis_name="s", num_cores=NC, num_subcores=NS
    )


def _make_deg(e):
    ept = e // NW
    nchunk = ept // CHUNK
    rpt = NPAD // NS

    @functools.partial(
        pl.kernel,
        out_type=jax.ShapeDtypeStruct((NC, NPAD), jnp.float32),
        mesh=_sc_mesh(),
        scratch_types=[
            pltpu.VMEM((nchunk, CHUNK), jnp.int32),
            pltpu.VMEM((CHUNK,), jnp.float32),
            pltpu.VMEM_SHARED((NPAD,), jnp.float32),
        ],
    )
    def deg_kernel(dst_hbm, ones_hbm, zeros_hbm, out_hbm, didx_v, ones_v, acc_sh):
        c = lax.axis_index("c")
        s = lax.axis_index("s")
        wid = c * NS + s
        base_r = s * rpt
        pltpu.sync_copy(zeros_hbm, acc_sh.at[pl.ds(base_r, rpt)])
        pltpu.sync_copy(ones_hbm, ones_v)
        pltpu.sync_copy(dst_hbm.at[wid], didx_v)
        plsc.subcore_barrier()

        def body(i, carry):
            pltpu.sync_copy(ones_v, acc_sh.at[didx_v.at[i]], add=True)
            return carry

        lax.fori_loop(0, nchunk, body, 0)
        plsc.subcore_barrier()
        pltpu.sync_copy(acc_sh.at[pl.ds(base_r, rpt)], out_hbm.at[c, pl.ds(base_r, rpt)])

    return deg_kernel


def _make_agg(e, d):
    ept = e // NW
    nchunk = ept // CHUNK
    rpt = NPAD // NS

    @functools.partial(
        pl.kernel,
        out_type=jax.ShapeDtypeStruct((NC, NPAD, d), jnp.float32),
        mesh=_sc_mesh(),
        scratch_types=[
            pltpu.VMEM((2, CHUNK), jnp.int32),
            pltpu.VMEM((nchunk, CHUNK), jnp.int32),
            pltpu.VMEM((2, CHUNK, d), jnp.float32),
            pltpu.VMEM_SHARED((NPAD, d), jnp.float32),
            pltpu.SemaphoreType.DMA,
            pltpu.SemaphoreType.DMA,
        ],
    )
    def agg_kernel(h_hbm, src_hbm, dst_hbm, zeros_hbm, out_hbm,
                   sidx_v, didx_v, rows_v, acc_sh, sem0, sem1):
        c = lax.axis_index("c")
        s = lax.axis_index("s")
        wid = c * NS + s
        base_r = s * rpt
        pltpu.sync_copy(zeros_hbm, acc_sh.at[pl.ds(base_r, rpt)])
        pltpu.sync_copy(dst_hbm.at[wid], didx_v)
        plsc.subcore_barrier()

        pltpu.sync_copy(src_hbm.at[wid, 0], sidx_v.at[0])
        pltpu.async_copy(h_hbm.at[sidx_v.at[0]], rows_v.at[0], sem0)

        def body(p, carry):
            i0 = 2 * p
            pltpu.sync_copy(src_hbm.at[wid, i0 + 1], sidx_v.at[1])
            pltpu.async_copy(h_hbm.at[sidx_v.at[1]], rows_v.at[1], sem1)
            pltpu.make_async_copy(h_hbm.at[sidx_v.at[0]], rows_v.at[0], sem0).wait()
            pltpu.sync_copy(rows_v.at[0], acc_sh.at[didx_v.at[i0]], add=True)

            @pl.when(i0 + 2 < nchunk)
            def _():
                pltpu.sync_copy(src_hbm.at[wid, i0 + 2], sidx_v.at[0])
                pltpu.async_copy(h_hbm.at[sidx_v.at[0]], rows_v.at[0], sem0)

            pltpu.make_async_copy(h_hbm.at[sidx_v.at[1]], rows_v.at[1], sem1).wait()
            pltpu.sync_copy(rows_v.at[1], acc_sh.at[didx_v.at[i0 + 1]], add=True)
            return carry

        lax.fori_loop(0, nchunk // 2, body, 0)
        if nchunk % 2 == 1:
            last = nchunk - 1
            pltpu.make_async_copy(h_hbm.at[sidx_v.at[0]], rows_v.at[0], sem0).wait()
            pltpu.sync_copy(rows_v.at[0], acc_sh.at[didx_v.at[last]], add=True)
        plsc.subcore_barrier()
        pltpu.sync_copy(acc_sh.at[pl.ds(base_r, rpt)], out_hbm.at[c, pl.ds(base_r, rpt)])

    return agg_kernel


def _tc_mm(x, W1):
    n = x.shape[0]

    def body(x_ref, w_ref, m_ref):
        m_ref[...] = jnp.dot(x_ref[...], w_ref[...], preferred_element_type=jnp.float32)

    return pl.pallas_call(
        body, out_shape=jax.ShapeDtypeStruct((n, W1.shape[1]), jnp.float32)
    )(x, W1)


def _tc_scale(cnt, m):
    n = m.shape[0]

    def body(cnt_ref, m_ref, h_ref, dis_ref):
        deg = 1.0 + cnt_ref[0, :n] + cnt_ref[1, :n]
        dis = lax.rsqrt(deg)[:, None]
        dis_ref[...] = dis
        h_ref[...] = m_ref[...] * dis

    return pl.pallas_call(
        body,
        out_shape=(
            jax.ShapeDtypeStruct((n, m.shape[1]), jnp.float32),
            jax.ShapeDtypeStruct((n, 1), jnp.float32),
        ),
    )(cnt, m)


def _tc_layer(p, hp, dis, b, g, be, Wn):
    n, _ = hp.shape
    dn = Wn.shape[1]

    def body(p_ref, hp_ref, dis_ref, b_ref, g_ref, be_ref, w_ref, o_ref):
        t = (p_ref[0, :n] + p_ref[1, :n] + hp_ref[...]) * dis_ref[...] + b_ref[...]
        mu = jnp.mean(t, axis=0, keepdims=True)
        dt = t - mu
        var = jnp.mean(dt * dt, axis=0, keepdims=True)
        z = g_ref[...] * dt * lax.rsqrt(var + 1e-5) + be_ref[...]
        z = jnp.maximum(z, 0.0)
        o_ref[...] = jnp.dot(z, w_ref[...], preferred_element_type=jnp.float32) * dis_ref[...]

    return pl.pallas_call(
        body, out_shape=jax.ShapeDtypeStruct((n, dn), jnp.float32)
    )(p, hp, dis, b, g, be, Wn)


def _tc_final(p, hp, dis, b, dout):
    n, _ = hp.shape

    def body(p_ref, hp_ref, dis_ref, b_ref, o_ref):
        t = (p_ref[0, :n] + p_ref[1, :n] + hp_ref[...]) * dis_ref[...] + b_ref[...]
        u = t[:, :dout]
        m = jnp.max(u, axis=1, keepdims=True)
        ex = jnp.exp(u - m)
        lse = jnp.log(jnp.sum(ex, axis=1, keepdims=True))
        o_ref[...] = u - m - lse

    return pl.pallas_call(
        body, out_shape=jax.ShapeDtypeStruct((n, dout), jnp.float32)
    )(p, hp, dis, b)


def kernel(x, edge_index, W1, b1, g1, be1, W2, b2, g2, be2, W3, b3):
    n = x.shape[0]
    e = edge_index.shape[1]
    dout = W3.shape[1]
    ept = e // NW
    nchunk = ept // CHUNK

    d3 = D
    ei = edge_index.astype(jnp.int32)
    src3 = ei[0].reshape(NW, nchunk, CHUNK)
    dst3 = ei[1].reshape(NW, nchunk, CHUNK)
    W3p = jnp.pad(W3, ((0, 0), (0, d3 - dout)))
    b3p = jnp.pad(b3, (0, d3 - dout))
    zrows = jnp.zeros((NPAD // NS, D), jnp.float32)
    zdeg = jnp.zeros((NPAD // NS,), jnp.float32)
    ones1 = jnp.ones((CHUNK,), jnp.float32)

    cnt = _make_deg(e)(dst3, ones1, zdeg)
    m1 = _tc_mm(x, W1)
    h1p, dis = _tc_scale(cnt, m1)

    agg = _make_agg(e, D)
    p1 = agg(h1p, src3, dst3, zrows)
    h2p = _tc_layer(p1, h1p, dis, b1[None], g1[None], be1[None], W2)
    p2 = agg(h2p, src3, dst3, zrows)
    h3p = _tc_layer(p2, h2p, dis, b2[None], g2[None], be2[None], W3p)
    p3 = agg(h3p, src3, dst3, zrows)
    return _tc_final(p3, h3p, dis, b3p[None], dout)
